# baseline (speedup 1.0000x reference)
import functools

import jax
import jax.numpy as jnp
from jax import lax
from jax.experimental import pallas as pl
from jax.experimental.pallas import tpu as pltpu
from jax.experimental.pallas import tpu_sc as plsc

S = 64
B = 64
H = 8
NPOS = 6
SIM_NB = 12
VAL_NB = 10
OUT_NB = 12

_R_CONN = 0
_R_SIMP = 8
_R_AGGP = 24
_R_OCONN = 56
_R_VCONN = 68
_R_VALP = 148
_R_OUTP = 404
_R_TOT = 532

_M1 = 0x55555555
_M2 = 0x33333333
_M4 = 0x0F0F0F0F
_MBYTE = 0x01010101


def _iota16():
    return lax.iota(jnp.int32, 16)


def _popcount2(x0, x1):
    def half(v):
        v = v - (jnp.right_shift(v, 1) & _M1)
        return (v & _M2) + (jnp.right_shift(v, 2) & _M2)
    s = half(x0) + half(x1)
    s = (s + jnp.right_shift(s, 4)) & _M4
    return jnp.right_shift(s * _MBYTE, 24)


def _fused_body(qk_hbm, tab_hbm, out_hbm,
                qk_v, conn_v, simp_v, vconn_v, valp_v, aggp_v,
                oconn_v, outp_v, attw_v, pw_v, pwall_v,
                agg_v, comb_v, res_v, proj_sh, comb_sh,
                sem_a, sem_b, sem_c):
    c = lax.axis_index("c")
    sid = lax.axis_index("s")
    h = sid // 2
    t = sid % 2
    rb = c + t * (3 - 2 * c)

    cp_simp = pltpu.async_copy(
        tab_hbm.at[pl.ds(_R_SIMP + 2 * h, 2)], simp_v, sem_a)
    cp_vconn = pltpu.async_copy(
        tab_hbm.at[pl.ds(_R_VCONN + 10 * h, 10), pl.ds(t * 32, 32)],
        vconn_v, sem_b)
    cp_valp = pltpu.async_copy(
        tab_hbm.at[pl.ds(_R_VALP + 32 * h + 16 * t, 16)], valp_v, sem_b)
    cp_aggp = pltpu.async_copy(
        tab_hbm.at[pl.ds(_R_AGGP + 4 * h, 4)], aggp_v, sem_c)
    cp_oconn = pltpu.async_copy(
        tab_hbm.at[pl.ds(_R_OCONN, OUT_NB)], oconn_v, sem_c)
    cp_outp = pltpu.async_copy(
        tab_hbm.at[pl.ds(_R_OUTP, 128)], outp_v, sem_c)
    pltpu.sync_copy(qk_hbm, qk_v)
    pltpu.sync_copy(tab_hbm.at[_R_CONN + h], conn_v)

    lanes = _iota16()
    zeros16 = jnp.zeros((16,), jnp.int32)

    cqi = conn_v[pl.ds(0, 16)]
    cqw = conn_v[pl.ds(16, 16)]
    cki = conn_v[pl.ds(32, 16)]
    ckw = conn_v[pl.ds(48, 16)]

    def _ortree(bits):
        while len(bits) > 1:
            bits = [bits[k] | bits[k + 1]
                    for k in range(0, len(bits) - 1, 2)] + \
                   (bits[-1:] if len(bits) % 2 else [])
        return bits[0]

    idx_i = lanes + rb * 16
    a_q = _ortree([
        plsc.load_gather(qk_v, [idx_i, jnp.full((16,), cqi[b], jnp.int32)])
        * cqw[b] for b in range(SIM_NB)])

    attw_v[0] = zeros16
    attw_v[1] = zeros16
    cp_simp.wait()

    for jb in range(4):
        @pl.when(jb <= rb)
        def _():
            idx_j = lanes + jb * 16
            akb = _ortree([
                plsc.load_gather(qk_v,
                                 [idx_j, jnp.full((16,), cki[b], jnp.int32)])
                * ckw[b] for b in range(SIM_NB)])
            w = jnp.zeros((16,), jnp.int32)
            for jj in range(16):
                j = jb * 16 + jj
                addr = a_q + akb[jj]
                word = plsc.load_gather(
                    simp_v,
                    [jnp.right_shift(addr, 11),
                     jnp.right_shift(addr, 5) & 63])
                bit = jnp.right_shift(word, addr & 31) & 1
                bit = jnp.where(idx_i >= j, bit, 0)
                w = w | jnp.left_shift(bit, jj + (jb % 2) * 16)
            attw_v[jb // 2] = attw_v[jb // 2] | w

    cp_vconn.wait()
    cp_valp.wait()
    for g in range(2):
        nrow = jnp.right_shift(lanes + g * 16, 1)
        ncol = ((lanes + g * 16) & 1) * 32
        vcols = [vconn_v[b, pl.ds(g * 16, 16)] for b in range(VAL_NB)]

        for wsel in range(2):
            w = jnp.zeros((16,), jnp.int32)
            for jj in range(32):
                j = wsel * 32 + jj
                rowj = jnp.full((16,), j, jnp.int32)
                addr = _ortree([jnp.left_shift(
                    plsc.load_gather(qk_v, [rowj, vcols[b]]),
                    VAL_NB - 1 - b) for b in range(VAL_NB)])
                word = plsc.load_gather(
                    valp_v, [nrow, ncol + jnp.right_shift(addr, 5)])
                bit = jnp.right_shift(word, addr & 31) & 1
                w = w | jnp.left_shift(bit, jj)
            pw_v[g, wsel] = w

    pltpu.sync_copy(pw_v, proj_sh.at[h, pl.ds(2 * t, 2)])
    cp_aggp.wait()
    cp_oconn.wait()
    cp_outp.wait()
    plsc.subcore_barrier()

    pltpu.sync_copy(proj_sh.at[h], pwall_v)

    av0 = attw_v[0]
    av1 = attw_v[1]
    natv = _popcount2(av0, av1)
    for g in range(4):
        pg0 = pwall_v[g, 0]
        pg1 = pwall_v[g, 1]
        for i in range(16):
            counts = _popcount2(av0[i] & pg0, av1[i] & pg1)
            word = plsc.load_gather(
                aggp_v, [jnp.right_shift(counts, 5), lanes + g * 16])
            bit = jnp.right_shift(word, counts & 31) & 1
            agg_v[i, pl.ds(g * 16, 16)] = jnp.where(natv[i] > 0, bit, zeros16)

    pltpu.sync_copy(agg_v, comb_sh.at[pl.ds(rb * 16, 16), pl.ds(h * 64, 64)])
    plsc.subcore_barrier()

    s8 = sid // 8
    row0 = 16 * (c + 2 * s8 * (1 - c)) + 2 * sid
    pltpu.sync_copy(comb_sh.at[pl.ds(row0, 2)], comb_v)

    ocols = [[oconn_v[b, pl.ds(nb * 16, 16)] for b in range(OUT_NB)]
             for nb in range(4)]
    for r in range(2):
        rowr = jnp.full((16,), r, jnp.int32)
        for nb in range(4):
            idx_n = lanes + nb * 16
            addr = _ortree([jnp.left_shift(
                plsc.load_gather(comb_v, [rowr, ocols[nb][b]]),
                OUT_NB - 1 - b) for b in range(OUT_NB)])
            wrd = jnp.right_shift(addr, 5)
            word = plsc.load_gather(
                outp_v,
                [jnp.left_shift(idx_n, 1) | jnp.right_shift(wrd, 6),
                 wrd & 63])
            res_v[r, pl.ds(nb * 16, 16)] = \
                jnp.right_shift(word, addr & 31) & 1
    pltpu.sync_copy(res_v, out_hbm.at[pl.ds(row0, 2)])


def _pack_bits(m):
    mm = m.astype(jnp.int32).reshape(m.shape[:-1] + (-1, 32))
    return jnp.sum(mm << jnp.arange(32, dtype=jnp.int32), axis=-1,
                   dtype=jnp.int32)


_MESH = plsc.VectorSubcoreMesh(core_axis_name="c", subcore_axis_name="s",
                               num_cores=2, num_subcores=16)

_PARAMS = pltpu.CompilerParams(use_tc_tiling_on_sc=False,
                               needs_layout_passes=False)

_fused = functools.partial(
    pl.kernel, _fused_body,
    out_type=jax.ShapeDtypeStruct((S, B), jnp.int32),
    mesh=_MESH,
    compiler_params=_PARAMS,
    scratch_types=[
        pltpu.VMEM((S, B + NPOS), jnp.int32),
        pltpu.VMEM((64,), jnp.int32),
        pltpu.VMEM((2, 64), jnp.int32),
        pltpu.VMEM((VAL_NB, 32), jnp.int32),
        pltpu.VMEM((16, 64), jnp.int32),
        pltpu.VMEM((4, B), jnp.int32),
        pltpu.VMEM((OUT_NB, B), jnp.int32),
        pltpu.VMEM((128, 64), jnp.int32),
        pltpu.VMEM((2, 16), jnp.int32),
        pltpu.VMEM((2, 2, 16), jnp.int32),
        pltpu.VMEM((4, 2, 16), jnp.int32),
        pltpu.VMEM((16, B), jnp.int32),
        pltpu.VMEM((2, H * B), jnp.int32),
        pltpu.VMEM((2, B), jnp.int32),
        pltpu.VMEM_SHARED((H, 4, 2, 16), jnp.int32),
        pltpu.VMEM_SHARED((S, H * B), jnp.int32),
        pltpu.SemaphoreType.DMA,
        pltpu.SemaphoreType.DMA,
        pltpu.SemaphoreType.DMA,
    ],
)()


def kernel(tokens, sim_conn, sim_mem, val_conn, val_mem, agg_mem, out_conn,
           out_mem):
    shifts = jnp.arange(NPOS - 1, -1, -1)
    pos = ((jnp.arange(S)[:, None] >> shifts[None, :]) & 1).astype(jnp.int32)
    qk70 = jnp.concatenate([tokens.astype(jnp.int32), pos], axis=1)

    c = sim_conn[:, 0, :].astype(jnp.int32)
    w = (1 << jnp.arange(SIM_NB - 1, -1, -1)).astype(jnp.int32)
    isq = (c < B) | ((c >= 2 * B) & (c < 2 * B + NPOS))
    cq = jnp.where(c < B, c, c - B)
    ck = jnp.where(c < 2 * B, c - B, c - (B + NPOS))
    conn_pack = jnp.stack([
        jnp.where(isq, cq, 0), jnp.where(isq, w, 0),
        jnp.where(isq, 0, ck), jnp.where(isq, 0, w),
    ], axis=1).astype(jnp.int32)
    conn_pack = jnp.pad(conn_pack, ((0, 0), (0, 0), (0, 16 - SIM_NB)))

    tab = jnp.concatenate([
        conn_pack.reshape(8, 64),
        _pack_bits(sim_mem[:, 0, :]).reshape(16, 64),
        _pack_bits(agg_mem).transpose(0, 2, 1).reshape(32, 64),
        out_conn.astype(jnp.int32).T,
        val_conn.astype(jnp.int32).transpose(0, 2, 1).reshape(80, 64),
        _pack_bits(val_mem).reshape(256, 64),
        _pack_bits(out_mem).reshape(128, 64),
    ], axis=0)

    return _fused(qk70, tab)


# speedup vs baseline: 1.0603x; 1.0603x over previous
import functools

import jax
import jax.numpy as jnp
from jax import lax
from jax.experimental import pallas as pl
from jax.experimental.pallas import tpu as pltpu
from jax.experimental.pallas import tpu_sc as plsc

S = 64
B = 64
H = 8
NPOS = 6
SIM_NB = 12
VAL_NB = 10
OUT_NB = 12

_R_CONN = 0
_R_SIMP = 8
_R_AGGP = 24
_R_OCONN = 56
_R_VCONN = 68
_R_VALP = 148
_R_OUTP = 404
_R_TOT = 532

_M1 = 0x55555555
_M2 = 0x33333333
_M4 = 0x0F0F0F0F
_MBYTE = 0x01010101


def _iota16():
    return lax.iota(jnp.int32, 16)


def _popcount2(x0, x1):
    def half(v):
        v = v - (jnp.right_shift(v, 1) & _M1)
        return (v & _M2) + (jnp.right_shift(v, 2) & _M2)
    s = half(x0) + half(x1)
    s = (s + jnp.right_shift(s, 4)) & _M4
    return jnp.right_shift(s * _MBYTE, 24)


def _fused_body(qk_hbm, tab_hbm, out_hbm,
                qk_v, conn_v, simp_v, vconn_v, valp_v, aggp_v,
                oconn_v, outp_v, attw_v, pw_v, pwall_v,
                agg_v, comb_v, res_v, proj_sh, comb_sh,
                sem_a, sem_b, sem_c):
    c = lax.axis_index("c")
    sid = lax.axis_index("s")
    h = sid // 2
    t = sid % 2
    rb = c + t * (3 - 2 * c)

    cp_simp = pltpu.async_copy(
        tab_hbm.at[pl.ds(_R_SIMP + 2 * h, 2)], simp_v, sem_a)
    cp_vconn = pltpu.async_copy(
        tab_hbm.at[pl.ds(_R_VCONN + 10 * h, 10), pl.ds(t * 32, 32)],
        vconn_v, sem_b)
    cp_valp = pltpu.async_copy(
        tab_hbm.at[pl.ds(_R_VALP + 32 * h + 16 * t, 16)], valp_v, sem_b)
    cp_aggp = pltpu.async_copy(
        tab_hbm.at[pl.ds(_R_AGGP + 4 * h, 4)], aggp_v, sem_c)
    cp_oconn = pltpu.async_copy(
        tab_hbm.at[pl.ds(_R_OCONN, OUT_NB)], oconn_v, sem_c)
    cp_outp = pltpu.async_copy(
        tab_hbm.at[pl.ds(_R_OUTP, 128)], outp_v, sem_c)
    pltpu.sync_copy(qk_hbm, qk_v)
    pltpu.sync_copy(tab_hbm.at[_R_CONN + h], conn_v)

    lanes = _iota16()
    zeros16 = jnp.zeros((16,), jnp.int32)

    cqi = conn_v[pl.ds(0, 16)]
    cqw = conn_v[pl.ds(16, 16)]
    cki = conn_v[pl.ds(32, 16)]
    ckw = conn_v[pl.ds(48, 16)]

    def _ortree(bits):
        while len(bits) > 1:
            bits = [bits[k] | bits[k + 1]
                    for k in range(0, len(bits) - 1, 2)] + \
                   (bits[-1:] if len(bits) % 2 else [])
        return bits[0]

    idx_i = lanes + rb * 16
    a_q = _ortree([
        plsc.load_gather(qk_v, [idx_i, jnp.full((16,), cqi[b], jnp.int32)])
        * cqw[b] for b in range(SIM_NB)])

    attw_v[0] = zeros16
    attw_v[1] = zeros16
    cp_simp.wait()

    for jb in range(4):
        @pl.when(jb <= rb)
        def _():
            idx_j = lanes + jb * 16
            akb = _ortree([
                plsc.load_gather(qk_v,
                                 [idx_j, jnp.full((16,), cki[b], jnp.int32)])
                * ckw[b] for b in range(SIM_NB)])
            w = jnp.zeros((16,), jnp.int32)
            for jj in range(16):
                j = jb * 16 + jj
                addr = a_q + akb[jj]
                word = plsc.load_gather(
                    simp_v,
                    [jnp.right_shift(addr, 11),
                     jnp.right_shift(addr, 5) & 63])
                bit = jnp.right_shift(word, addr & 31) & 1
                bit = jnp.where(idx_i >= j, bit, 0)
                w = w | jnp.left_shift(bit, jj + (jb % 2) * 16)
            attw_v[jb // 2] = attw_v[jb // 2] | w

    cp_vconn.wait()
    cp_valp.wait()
    for g in range(2):
        nrow = jnp.right_shift(lanes + g * 16, 1)
        ncol = ((lanes + g * 16) & 1) * 32
        vcols = [vconn_v[b, pl.ds(g * 16, 16)] for b in range(VAL_NB)]

        def proj_j(j, w, base):
            rowj = jnp.full((16,), j + base, jnp.int32)
            addr = _ortree([jnp.left_shift(
                plsc.load_gather(qk_v, [rowj, vcols[b]]),
                VAL_NB - 1 - b) for b in range(VAL_NB)])
            word = plsc.load_gather(
                valp_v, [nrow, ncol + jnp.right_shift(addr, 5)])
            bit = jnp.right_shift(word, addr & 31) & 1
            return w | jnp.left_shift(bit, j)

        pw_v[g, 0] = lax.fori_loop(0, 32, lambda j, w: proj_j(j, w, 0),
                                   jnp.zeros((16,), jnp.int32))
        pw_v[g, 1] = lax.fori_loop(0, 32, lambda j, w: proj_j(j, w, 32),
                                   jnp.zeros((16,), jnp.int32))

    pltpu.sync_copy(pw_v, proj_sh.at[h, pl.ds(2 * t, 2)])
    cp_aggp.wait()
    cp_oconn.wait()
    cp_outp.wait()
    plsc.subcore_barrier()

    pltpu.sync_copy(proj_sh.at[h], pwall_v)

    av0 = attw_v[0]
    av1 = attw_v[1]
    natv = _popcount2(av0, av1)
    for g in range(4):
        pg0 = pwall_v[g, 0]
        pg1 = pwall_v[g, 1]
        for i in range(16):
            counts = _popcount2(av0[i] & pg0, av1[i] & pg1)
            word = plsc.load_gather(
                aggp_v, [jnp.right_shift(counts, 5), lanes + g * 16])
            bit = jnp.right_shift(word, counts & 31) & 1
            agg_v[i, pl.ds(g * 16, 16)] = jnp.where(natv[i] > 0, bit, zeros16)

    pltpu.sync_copy(agg_v, comb_sh.at[pl.ds(rb * 16, 16), pl.ds(h * 64, 64)])
    plsc.subcore_barrier()

    s8 = sid // 8
    row0 = 16 * (c + 2 * s8 * (1 - c)) + 2 * sid
    pltpu.sync_copy(comb_sh.at[pl.ds(row0, 2)], comb_v)

    ocols = [[oconn_v[b, pl.ds(nb * 16, 16)] for b in range(OUT_NB)]
             for nb in range(4)]
    for r in range(2):
        rowr = jnp.full((16,), r, jnp.int32)
        for nb in range(4):
            idx_n = lanes + nb * 16
            addr = _ortree([jnp.left_shift(
                plsc.load_gather(comb_v, [rowr, ocols[nb][b]]),
                OUT_NB - 1 - b) for b in range(OUT_NB)])
            wrd = jnp.right_shift(addr, 5)
            word = plsc.load_gather(
                outp_v,
                [jnp.left_shift(idx_n, 1) | jnp.right_shift(wrd, 6),
                 wrd & 63])
            res_v[r, pl.ds(nb * 16, 16)] = \
                jnp.right_shift(word, addr & 31) & 1
    pltpu.sync_copy(res_v, out_hbm.at[pl.ds(row0, 2)])


def _pack_bits(m):
    mm = m.astype(jnp.int32).reshape(m.shape[:-1] + (-1, 32))
    return jnp.sum(mm << jnp.arange(32, dtype=jnp.int32), axis=-1,
                   dtype=jnp.int32)


_MESH = plsc.VectorSubcoreMesh(core_axis_name="c", subcore_axis_name="s",
                               num_cores=2, num_subcores=16)

_PARAMS = pltpu.CompilerParams(use_tc_tiling_on_sc=False,
                               needs_layout_passes=False)

_fused = functools.partial(
    pl.kernel, _fused_body,
    out_type=jax.ShapeDtypeStruct((S, B), jnp.int32),
    mesh=_MESH,
    compiler_params=_PARAMS,
    scratch_types=[
        pltpu.VMEM((S, B + NPOS), jnp.int32),
        pltpu.VMEM((64,), jnp.int32),
        pltpu.VMEM((2, 64), jnp.int32),
        pltpu.VMEM((VAL_NB, 32), jnp.int32),
        pltpu.VMEM((16, 64), jnp.int32),
        pltpu.VMEM((4, B), jnp.int32),
        pltpu.VMEM((OUT_NB, B), jnp.int32),
        pltpu.VMEM((128, 64), jnp.int32),
        pltpu.VMEM((2, 16), jnp.int32),
        pltpu.VMEM((2, 2, 16), jnp.int32),
        pltpu.VMEM((4, 2, 16), jnp.int32),
        pltpu.VMEM((16, B), jnp.int32),
        pltpu.VMEM((2, H * B), jnp.int32),
        pltpu.VMEM((2, B), jnp.int32),
        pltpu.VMEM_SHARED((H, 4, 2, 16), jnp.int32),
        pltpu.VMEM_SHARED((S, H * B), jnp.int32),
        pltpu.SemaphoreType.DMA,
        pltpu.SemaphoreType.DMA,
        pltpu.SemaphoreType.DMA,
    ],
)()


def kernel(tokens, sim_conn, sim_mem, val_conn, val_mem, agg_mem, out_conn,
           out_mem):
    shifts = jnp.arange(NPOS - 1, -1, -1)
    pos = ((jnp.arange(S)[:, None] >> shifts[None, :]) & 1).astype(jnp.int32)
    qk70 = jnp.concatenate([tokens.astype(jnp.int32), pos], axis=1)

    c = sim_conn[:, 0, :].astype(jnp.int32)
    w = (1 << jnp.arange(SIM_NB - 1, -1, -1)).astype(jnp.int32)
    isq = (c < B) | ((c >= 2 * B) & (c < 2 * B + NPOS))
    cq = jnp.where(c < B, c, c - B)
    ck = jnp.where(c < 2 * B, c - B, c - (B + NPOS))
    conn_pack = jnp.stack([
        jnp.where(isq, cq, 0), jnp.where(isq, w, 0),
        jnp.where(isq, 0, ck), jnp.where(isq, 0, w),
    ], axis=1).astype(jnp.int32)
    conn_pack = jnp.pad(conn_pack, ((0, 0), (0, 0), (0, 16 - SIM_NB)))

    tab = jnp.concatenate([
        conn_pack.reshape(8, 64),
        _pack_bits(sim_mem[:, 0, :]).reshape(16, 64),
        _pack_bits(agg_mem).transpose(0, 2, 1).reshape(32, 64),
        out_conn.astype(jnp.int32).T,
        val_conn.astype(jnp.int32).transpose(0, 2, 1).reshape(80, 64),
        _pack_bits(val_mem).reshape(256, 64),
        _pack_bits(out_mem).reshape(128, 64),
    ], axis=0)

    return _fused(qk70, tab)


# speedup vs baseline: 1.0650x; 1.0044x over previous
import functools

import jax
import jax.numpy as jnp
from jax import lax
from jax.experimental import pallas as pl
from jax.experimental.pallas import tpu as pltpu
from jax.experimental.pallas import tpu_sc as plsc

S = 64
B = 64
H = 8
NPOS = 6
SIM_NB = 12
VAL_NB = 10
OUT_NB = 12

_R_CONN = 0
_R_SIMP = 8
_R_AGGP = 24
_R_OCONN = 56
_R_VCONN = 68
_R_VALP = 148
_R_OUTP = 404
_R_TOT = 532

_M1 = 0x55555555
_M2 = 0x33333333
_M4 = 0x0F0F0F0F
_MBYTE = 0x01010101


def _iota16():
    return lax.iota(jnp.int32, 16)


def _popcount2(x0, x1):
    def half(v):
        v = v - (jnp.right_shift(v, 1) & _M1)
        return (v & _M2) + (jnp.right_shift(v, 2) & _M2)
    s = half(x0) + half(x1)
    s = (s + jnp.right_shift(s, 4)) & _M4
    return jnp.right_shift(s * _MBYTE, 24)


def _fused_body(qk_hbm, tab_hbm, out_hbm,
                qk_v, conn_v, simp_v, vconn_v, valp_v, aggp_v,
                oconn_v, outp_v, attw_v, pw_v, pwall_v,
                agg_v, comb_v, res_v, proj_sh, comb_sh,
                sem_a, sem_b, sem_c):
    c = lax.axis_index("c")
    sid = lax.axis_index("s")
    h = sid // 2
    t = sid % 2
    rb = c + t * (3 - 2 * c)

    cp_simp = pltpu.async_copy(
        tab_hbm.at[pl.ds(_R_SIMP + 2 * h, 2)], simp_v, sem_a)
    cp_vconn = pltpu.async_copy(
        tab_hbm.at[pl.ds(_R_VCONN + 10 * h, 10), pl.ds(t * 32, 32)],
        vconn_v, sem_b)
    cp_valp = pltpu.async_copy(
        tab_hbm.at[pl.ds(_R_VALP + 32 * h + 16 * t, 16)], valp_v, sem_b)
    cp_aggp = pltpu.async_copy(
        tab_hbm.at[pl.ds(_R_AGGP + 4 * h, 4)], aggp_v, sem_c)
    cp_oconn = pltpu.async_copy(
        tab_hbm.at[pl.ds(_R_OCONN, OUT_NB)], oconn_v, sem_c)
    cp_outp = pltpu.async_copy(
        tab_hbm.at[pl.ds(_R_OUTP, 128)], outp_v, sem_c)
    pltpu.sync_copy(qk_hbm, qk_v)
    pltpu.sync_copy(tab_hbm.at[_R_CONN + h], conn_v)

    lanes = _iota16()
    zeros16 = jnp.zeros((16,), jnp.int32)

    cqi = conn_v[pl.ds(0, 16)]
    cqw = conn_v[pl.ds(16, 16)]
    cki = conn_v[pl.ds(32, 16)]
    ckw = conn_v[pl.ds(48, 16)]

    def _ortree(bits):
        while len(bits) > 1:
            bits = [bits[k] | bits[k + 1]
                    for k in range(0, len(bits) - 1, 2)] + \
                   (bits[-1:] if len(bits) % 2 else [])
        return bits[0]

    idx_i = lanes + rb * 16
    a_q = _ortree([
        plsc.load_gather(qk_v, [idx_i, jnp.full((16,), cqi[b], jnp.int32)])
        * cqw[b] for b in range(SIM_NB)])

    attw_v[0] = zeros16
    attw_v[1] = zeros16
    cp_simp.wait()

    for jb in range(4):
        @pl.when(jb <= rb)
        def _():
            idx_j = lanes + jb * 16
            akb = _ortree([
                plsc.load_gather(qk_v,
                                 [idx_j, jnp.full((16,), cki[b], jnp.int32)])
                * ckw[b] for b in range(SIM_NB)])
            w = jnp.zeros((16,), jnp.int32)
            for jj in range(16):
                j = jb * 16 + jj
                addr = a_q + akb[jj]
                word = plsc.load_gather(
                    simp_v,
                    [jnp.right_shift(addr, 11),
                     jnp.right_shift(addr, 5) & 63])
                bit = jnp.right_shift(word, addr & 31) & 1
                bit = jnp.where(idx_i >= j, bit, 0)
                w = w | jnp.left_shift(bit, jj + (jb % 2) * 16)
            attw_v[jb // 2] = attw_v[jb // 2] | w

    cp_vconn.wait()
    cp_valp.wait()
    for g in range(2):
        nrow = jnp.right_shift(lanes + g * 16, 1)
        ncol = ((lanes + g * 16) & 1) * 32
        vcols = [vconn_v[b, pl.ds(g * 16, 16)] for b in range(VAL_NB)]
        planes = [[plsc.load_gather(
                       qk_v, [jnp.full((16,), S + ws, jnp.int32), vcols[b]])
                   for b in range(VAL_NB)] for ws in range(2)]

        for ws in range(2):
            pp = planes[ws]

            def proj_j(j, w):
                addr = _ortree([jnp.left_shift(
                    jnp.right_shift(pp[b], j) & 1,
                    VAL_NB - 1 - b) for b in range(VAL_NB)])
                word = plsc.load_gather(
                    valp_v, [nrow, ncol + jnp.right_shift(addr, 5)])
                bit = jnp.right_shift(word, addr & 31) & 1
                return w | jnp.left_shift(bit, j)

            pw_v[g, ws] = lax.fori_loop(0, 32, proj_j,
                                        jnp.zeros((16,), jnp.int32))

    pltpu.sync_copy(pw_v, proj_sh.at[h, pl.ds(2 * t, 2)])
    cp_aggp.wait()
    cp_oconn.wait()
    cp_outp.wait()
    plsc.subcore_barrier()

    pltpu.sync_copy(proj_sh.at[h], pwall_v)

    av0 = attw_v[0]
    av1 = attw_v[1]
    natv = _popcount2(av0, av1)
    for g in range(4):
        pg0 = pwall_v[g, 0]
        pg1 = pwall_v[g, 1]
        for i in range(16):
            counts = _popcount2(av0[i] & pg0, av1[i] & pg1)
            word = plsc.load_gather(
                aggp_v, [jnp.right_shift(counts, 5), lanes + g * 16])
            bit = jnp.right_shift(word, counts & 31) & 1
            agg_v[i, pl.ds(g * 16, 16)] = jnp.where(natv[i] > 0, bit, zeros16)

    pltpu.sync_copy(agg_v, comb_sh.at[pl.ds(rb * 16, 16), pl.ds(h * 64, 64)])
    plsc.subcore_barrier()

    s8 = sid // 8
    row0 = 16 * (c + 2 * s8 * (1 - c)) + 2 * sid
    pltpu.sync_copy(comb_sh.at[pl.ds(row0, 2)], comb_v)

    ocols = [[oconn_v[b, pl.ds(nb * 16, 16)] for b in range(OUT_NB)]
             for nb in range(4)]
    for r in range(2):
        rowr = jnp.full((16,), r, jnp.int32)
        for nb in range(4):
            idx_n = lanes + nb * 16
            addr = _ortree([jnp.left_shift(
                plsc.load_gather(comb_v, [rowr, ocols[nb][b]]),
                OUT_NB - 1 - b) for b in range(OUT_NB)])
            wrd = jnp.right_shift(addr, 5)
            word = plsc.load_gather(
                outp_v,
                [jnp.left_shift(idx_n, 1) | jnp.right_shift(wrd, 6),
                 wrd & 63])
            res_v[r, pl.ds(nb * 16, 16)] = \
                jnp.right_shift(word, addr & 31) & 1
    pltpu.sync_copy(res_v, out_hbm.at[pl.ds(row0, 2)])


def _pack_bits(m):
    mm = m.astype(jnp.int32).reshape(m.shape[:-1] + (-1, 32))
    return jnp.sum(mm << jnp.arange(32, dtype=jnp.int32), axis=-1,
                   dtype=jnp.int32)


_MESH = plsc.VectorSubcoreMesh(core_axis_name="c", subcore_axis_name="s",
                               num_cores=2, num_subcores=16)

_PARAMS = pltpu.CompilerParams(use_tc_tiling_on_sc=False,
                               needs_layout_passes=False)

_fused = functools.partial(
    pl.kernel, _fused_body,
    out_type=jax.ShapeDtypeStruct((S, B), jnp.int32),
    mesh=_MESH,
    compiler_params=_PARAMS,
    scratch_types=[
        pltpu.VMEM((S + 2, B + NPOS), jnp.int32),
        pltpu.VMEM((64,), jnp.int32),
        pltpu.VMEM((2, 64), jnp.int32),
        pltpu.VMEM((VAL_NB, 32), jnp.int32),
        pltpu.VMEM((16, 64), jnp.int32),
        pltpu.VMEM((4, B), jnp.int32),
        pltpu.VMEM((OUT_NB, B), jnp.int32),
        pltpu.VMEM((128, 64), jnp.int32),
        pltpu.VMEM((2, 16), jnp.int32),
        pltpu.VMEM((2, 2, 16), jnp.int32),
        pltpu.VMEM((4, 2, 16), jnp.int32),
        pltpu.VMEM((16, B), jnp.int32),
        pltpu.VMEM((2, H * B), jnp.int32),
        pltpu.VMEM((2, B), jnp.int32),
        pltpu.VMEM_SHARED((H, 4, 2, 16), jnp.int32),
        pltpu.VMEM_SHARED((S, H * B), jnp.int32),
        pltpu.SemaphoreType.DMA,
        pltpu.SemaphoreType.DMA,
        pltpu.SemaphoreType.DMA,
    ],
)()


def kernel(tokens, sim_conn, sim_mem, val_conn, val_mem, agg_mem, out_conn,
           out_mem):
    shifts = jnp.arange(NPOS - 1, -1, -1)
    pos = ((jnp.arange(S)[:, None] >> shifts[None, :]) & 1).astype(jnp.int32)
    qk70 = jnp.concatenate([tokens.astype(jnp.int32), pos], axis=1)
    qk70 = jnp.concatenate([qk70, _pack_bits(qk70.T).T], axis=0)

    c = sim_conn[:, 0, :].astype(jnp.int32)
    w = (1 << jnp.arange(SIM_NB - 1, -1, -1)).astype(jnp.int32)
    isq = (c < B) | ((c >= 2 * B) & (c < 2 * B + NPOS))
    cq = jnp.where(c < B, c, c - B)
    ck = jnp.where(c < 2 * B, c - B, c - (B + NPOS))
    conn_pack = jnp.stack([
        jnp.where(isq, cq, 0), jnp.where(isq, w, 0),
        jnp.where(isq, 0, ck), jnp.where(isq, 0, w),
    ], axis=1).astype(jnp.int32)
    conn_pack = jnp.pad(conn_pack, ((0, 0), (0, 0), (0, 16 - SIM_NB)))

    tab = jnp.concatenate([
        conn_pack.reshape(8, 64),
        _pack_bits(sim_mem[:, 0, :]).reshape(16, 64),
        _pack_bits(agg_mem).transpose(0, 2, 1).reshape(32, 64),
        out_conn.astype(jnp.int32).T,
        val_conn.astype(jnp.int32).transpose(0, 2, 1).reshape(80, 64),
        _pack_bits(val_mem).reshape(256, 64),
        _pack_bits(out_mem).reshape(128, 64),
    ], axis=0)

    return _fused(qk70, tab)
